# baseline (speedup 1.0000x reference)
import functools

import jax
import jax.numpy as jnp
from jax import lax
from jax.experimental import pallas as pl
from jax.experimental.pallas import tpu as pltpu, tpu_sc as plsc

_PERIODS = (24, 7, 31, 12, 366, 1440)
_D = 16
_B = 16384
_NC = 2
_NS = 16
_NW = _NC * _NS
_BPW = _B // _NW
_NF = 6
_DOUT = _NF * _D
_DPITCH = 128
_GRP = _BPW // 16
_OFFS = tuple(sum(_PERIODS[:f]) for f in range(_NF))
_TWORDS = sum(_PERIODS) * _D


def _body(i0, i1, i2, i3, i4, i5, tab_hbm, out_hbm,
          idx_v, tab_v, rows_v, sem):
    idx_hbm = (i0, i1, i2, i3, i4, i5)
    wid = lax.axis_index("s") * _NC + lax.axis_index("c")
    copies = [pltpu.async_copy(tab_hbm, tab_v, sem)]
    for f in range(_NF):
        copies.append(pltpu.async_copy(
            idx_hbm[f].at[pl.ds(wid * _BPW, _BPW)], idx_v.at[f], sem))
    for cp in copies:
        cp.wait()

    lanes = lax.iota(jnp.int32, 16)
    skews = [(lanes + j) & 15 for j in range(_D)]

    @plsc.parallel_loop(0, _GRP, 1, unroll=4)
    def group(g):
        rowbase = (jnp.full((16,), g * 16, jnp.int32) + lanes) * _DPITCH
        for f in range(_NF):
            idxv = idx_v[f, pl.ds(g * 16, 16)]
            gbase = idxv * _D + jnp.full((16,), _OFFS[f] * _D, jnp.int32)
            sbase = rowbase + jnp.full((16,), f * _D, jnp.int32)
            for j in range(_D):
                x = plsc.load_gather(tab_v, [gbase + skews[j]])
                plsc.store_scatter(rows_v, [sbase + skews[j]], x)
    pltpu.sync_copy(rows_v, out_hbm.at[pl.ds(wid * _BPW * _DPITCH,
                                             _BPW * _DPITCH)])


@jax.jit
def _encode(i0, i1, i2, i3, i4, i5, tab_flat):
    mesh = plsc.VectorSubcoreMesh(core_axis_name="c", subcore_axis_name="s")
    kern = pl.kernel(
        _body,
        out_type=jax.ShapeDtypeStruct((_B * _DPITCH,), jnp.float32),
        mesh=mesh,
        scratch_types=[
            pltpu.VMEM((_NF, _BPW), jnp.int32),
            pltpu.VMEM((_TWORDS,), jnp.float32),
            pltpu.VMEM((_BPW * _DPITCH,), jnp.float32),
            pltpu.SemaphoreType.DMA,
        ],
        compiler_params=pltpu.CompilerParams(
            use_tc_tiling_on_sc=False, needs_layout_passes=False),
    )
    out = kern(i0, i1, i2, i3, i4, i5, tab_flat)
    return out.reshape(_B, _DPITCH)[:, :_DOUT]


def kernel(hour, day_of_week, day_of_month, month, day_of_year, minute_of_day,
           W_hour, W_day_of_week, W_day_of_month, W_month, W_day_of_year,
           W_minute_of_day):
    tab_flat = jnp.concatenate([
        w.reshape(-1)
        for w in (W_hour, W_day_of_week, W_day_of_month, W_month,
                  W_day_of_year, W_minute_of_day)
    ])
    return _encode(hour, day_of_week, day_of_month, month, day_of_year,
                   minute_of_day, tab_flat)


# speedup vs baseline: 1.0642x; 1.0642x over previous
import functools

import jax
import jax.numpy as jnp
from jax import lax
from jax.experimental import pallas as pl
from jax.experimental.pallas import tpu as pltpu, tpu_sc as plsc

_PERIODS = (24, 7, 31, 12, 366, 1440)
_D = 16
_B = 16384
_NC = 2
_NS = 16
_NW = _NC * _NS
_BPW = _B // _NW
_NF = 6
_DOUT = _NF * _D
_DPITCH = 128
_GRP = _BPW // 16
_OFFS = tuple(sum(_PERIODS[:f]) for f in range(_NF))
_TWORDS = sum(_PERIODS) * _D


def _body(i0, i1, i2, i3, i4, i5, tab_hbm, out_hbm,
          idx_v, tab_v, rows_v, sem):
    idx_hbm = (i0, i1, i2, i3, i4, i5)
    wid = lax.axis_index("s") * _NC + lax.axis_index("c")
    copies = [pltpu.async_copy(tab_hbm, tab_v, sem)]
    for f in range(_NF):
        copies.append(pltpu.async_copy(
            idx_hbm[f].at[pl.ds(wid * _BPW, _BPW)], idx_v.at[f], sem))
    for cp in copies:
        cp.wait()

    lanes = lax.iota(jnp.int32, 16)
    skews = [(lanes + j) & 15 for j in range(_D)]

    @plsc.parallel_loop(0, _GRP, 1, unroll=2)
    def group(g):
        rowbase = (jnp.full((16,), g * 16, jnp.int32) + lanes) * _DPITCH
        for f in range(_NF):
            idxv = idx_v[f, pl.ds(g * 16, 16)]
            gbase = idxv * _D + jnp.full((16,), _OFFS[f] * _D, jnp.int32)
            sbase = rowbase + jnp.full((16,), f * _D, jnp.int32)
            for j in range(_D):
                x = plsc.load_gather(tab_v, [gbase + skews[j]])
                plsc.store_scatter(rows_v, [sbase + skews[j]], x)
    pltpu.sync_copy(rows_v, out_hbm.at[pl.ds(wid * _BPW * _DPITCH,
                                             _BPW * _DPITCH)])


@jax.jit
def _encode(i0, i1, i2, i3, i4, i5, tab_flat):
    mesh = plsc.VectorSubcoreMesh(core_axis_name="c", subcore_axis_name="s")
    kern = pl.kernel(
        _body,
        out_type=jax.ShapeDtypeStruct((_B * _DPITCH,), jnp.float32),
        mesh=mesh,
        scratch_types=[
            pltpu.VMEM((_NF, _BPW), jnp.int32),
            pltpu.VMEM((_TWORDS,), jnp.float32),
            pltpu.VMEM((_BPW * _DPITCH,), jnp.float32),
            pltpu.SemaphoreType.DMA,
        ],
        compiler_params=pltpu.CompilerParams(
            use_tc_tiling_on_sc=False, needs_layout_passes=False),
    )
    out = kern(i0, i1, i2, i3, i4, i5, tab_flat)
    return out.reshape(_B, _DPITCH)[:, :_DOUT]


def kernel(hour, day_of_week, day_of_month, month, day_of_year, minute_of_day,
           W_hour, W_day_of_week, W_day_of_month, W_month, W_day_of_year,
           W_minute_of_day):
    tab_flat = jnp.concatenate([
        w.reshape(-1)
        for w in (W_hour, W_day_of_week, W_day_of_month, W_month,
                  W_day_of_year, W_minute_of_day)
    ])
    return _encode(hour, day_of_week, day_of_month, month, day_of_year,
                   minute_of_day, tab_flat)
